# baseline (speedup 1.0000x reference)
import functools

import jax
import jax.numpy as jnp
from jax import lax
from jax.experimental import pallas as pl
from jax.experimental.pallas import tpu as pltpu
from jax.experimental.pallas import tpu_sc as plsc

E = 8
K = 2
D = 768
F = 3072
N = 2048
S = N * K
B = 256
NB = S // B + E
PTOT = NB * B
FB = 768
NF = F // FB

NC, NS = 2, 16
NW = NC * NS


def _cumsum_rows(x, n):
    c = x
    sh = 1
    while sh < n:
        c = c + jnp.concatenate(
            [jnp.zeros((sh, x.shape[1]), x.dtype), c[:-sh]], axis=0)
        sh *= 2
    return c


def _cumsum_lanes(x, n):
    c = x
    sh = 1
    while sh < n:
        c = c + jnp.concatenate(
            [jnp.zeros((x.shape[0], sh), x.dtype), c[:, :-sh]], axis=1)
        sh *= 2
    return c


def _gate_body(x_ref, wg_ref, bg_ref, gs_ref, pp_ref, be_ref, ba_ref):
    logits = jnp.dot(x_ref[...], wg_ref[...],
                     preferred_element_type=jnp.float32) + bg_ref[...]
    col = lax.broadcasted_iota(jnp.int32, (N, E), 1)
    v0 = jnp.max(logits, axis=1, keepdims=True)
    i0 = jnp.min(jnp.where(logits == v0, col, E), axis=1, keepdims=True)
    masked = jnp.where(col == i0, -jnp.inf, logits)
    v1 = jnp.max(masked, axis=1, keepdims=True)
    i1 = jnp.min(jnp.where(masked == v1, col, E), axis=1, keepdims=True)
    e = jnp.exp(v1 - v0)
    s0 = 1.0 / (1.0 + e)
    gs_ref[...] = jnp.concatenate([s0, 1.0 - s0], axis=1)

    oh0 = (col == i0).astype(jnp.int32)
    oh1 = (col == i1).astype(jnp.int32)
    both = oh0 + oh1
    tot = jnp.sum(both, axis=0, keepdims=True)
    cex = _cumsum_rows(both, N) - both
    blocks_per_e = (tot + B - 1) // B
    cumb = _cumsum_lanes(blocks_per_e, E)
    p_off = jnp.concatenate(
        [jnp.zeros((1, 1), jnp.int32), cumb[:, :-1]], axis=1) * B
    rank0 = jnp.sum(jnp.where(col == i0, cex, 0), axis=1, keepdims=True)
    rank1 = jnp.sum(jnp.where(col == i1, cex, 0), axis=1, keepdims=True)
    off0 = jnp.sum(jnp.where(col == i0, p_off, 0), axis=1, keepdims=True)
    off1 = jnp.sum(jnp.where(col == i1, p_off, 0), axis=1, keepdims=True)
    pp_ref[...] = jnp.concatenate([off0 + rank0, off1 + rank1], axis=1)

    num_active = cumb[0, E - 1]
    gcol = lax.broadcasted_iota(jnp.int32, (NB, E), 0)
    be_raw = jnp.sum((gcol >= jnp.broadcast_to(cumb, (NB, E))).astype(
        jnp.int32), axis=1, keepdims=True)
    ecol = lax.broadcasted_iota(jnp.int32, (NB, E), 1)
    last_e = jnp.max(jnp.where(jnp.broadcast_to(tot, (NB, E)) > 0, ecol, 0),
                     axis=1, keepdims=True)
    gid = lax.broadcasted_iota(jnp.int32, (NB, 1), 0)
    be_ref[...] = jnp.where(gid < num_active,
                            jnp.minimum(be_raw, E - 1), last_e)
    ba_ref[...] = (gid < num_active).astype(jnp.int32)


def _gate(inp, w_gate, b_gate):
    return pl.pallas_call(
        _gate_body,
        out_shape=(jax.ShapeDtypeStruct((N, K), jnp.float32),
                   jax.ShapeDtypeStruct((N, K), jnp.int32),
                   jax.ShapeDtypeStruct((NB, 1), jnp.int32),
                   jax.ShapeDtypeStruct((NB, 1), jnp.int32)),
    )(inp, w_gate, b_gate.reshape(1, E))


_CH = 64


_TPW = N // NW


@functools.cache
def _make_dispatch():
    @functools.partial(
        pl.kernel,
        out_type=jax.ShapeDtypeStruct((PTOT, D), jnp.float32),
        mesh=plsc.VectorSubcoreMesh(core_axis_name="c", subcore_axis_name="s",
                                    num_cores=NC, num_subcores=NS),
        scratch_types=[
            pltpu.VMEM((_TPW,), jnp.int32),
            pltpu.VMEM((_TPW,), jnp.int32),
            pltpu.VMEM((_TPW, D), jnp.float32),
            pltpu.SemaphoreType.DMA,
        ],
    )
    def _dispatch(inp_h, pe_h, po_h, x_h, pe_v, po_v, rows_v, sem):
        wid = lax.axis_index("s") * NC + lax.axis_index("c")
        base = wid * _TPW
        pltpu.sync_copy(inp_h.at[pl.ds(base, _TPW)], rows_v)
        pltpu.sync_copy(pe_h.at[pl.ds(base, _TPW)], pe_v)
        pltpu.sync_copy(po_h.at[pl.ds(base, _TPW)], po_v)
        a = pltpu.async_copy(rows_v, x_h.at[pe_v], sem)
        b = pltpu.async_copy(rows_v, x_h.at[po_v], sem)
        a.wait()
        b.wait()

    return _dispatch


@functools.cache
def _make_combine_gather():
    @functools.partial(
        pl.kernel,
        out_type=jax.ShapeDtypeStruct((N, 2 * D), jnp.float32),
        mesh=plsc.VectorSubcoreMesh(core_axis_name="c", subcore_axis_name="s",
                                    num_cores=NC, num_subcores=NS),
        scratch_types=[
            pltpu.VMEM((_TPW,), jnp.int32),
            pltpu.VMEM((_TPW,), jnp.int32),
            pltpu.VMEM((_TPW, D), jnp.float32),
            pltpu.VMEM((_TPW, D), jnp.float32),
            pltpu.SemaphoreType.DMA,
        ],
    )
    def _cgather(y_h, pe_h, po_h, out_h, pe_v, po_v, a_v, b_v, sem):
        wid = lax.axis_index("s") * NC + lax.axis_index("c")
        tb = wid * _TPW
        pltpu.sync_copy(pe_h.at[pl.ds(tb, _TPW)], pe_v)
        pltpu.sync_copy(po_h.at[pl.ds(tb, _TPW)], po_v)
        a = pltpu.async_copy(y_h.at[pe_v], a_v, sem)
        b = pltpu.async_copy(y_h.at[po_v], b_v, sem)
        a.wait()
        b.wait()
        pltpu.sync_copy(a_v, out_h.at[pl.ds(tb, _TPW), pl.ds(0, D)])
        pltpu.sync_copy(b_v, out_h.at[pl.ds(tb, _TPW), pl.ds(D, D)])

    return _cgather


def _ffn_body(be_ref, ba_ref, x_ref, w1_ref, b1_ref, w2_ref, b2_ref, y_ref):
    g = pl.program_id(0)

    @pl.when(ba_ref[g] == 1)
    def _():
        h = jnp.dot(x_ref[...].astype(jnp.bfloat16),
                    w1_ref[0].astype(jnp.bfloat16),
                    preferred_element_type=jnp.float32) + b1_ref[0, 0, :]
        y = jnp.dot(jax.nn.gelu(h).astype(jnp.bfloat16),
                    w2_ref[0].astype(jnp.bfloat16),
                    preferred_element_type=jnp.float32)
        y_ref[...] = y + b2_ref[0, 0, :]


def _ffn(x_sorted, W1, b1, W2, b2, block_e, block_a):
    grid_spec = pltpu.PrefetchScalarGridSpec(
        num_scalar_prefetch=2,
        grid=(NB,),
        in_specs=[
            pl.BlockSpec((B, D), lambda g, be, ba: (g, 0)),
            pl.BlockSpec((1, D, F), lambda g, be, ba: (be[g], 0, 0)),
            pl.BlockSpec((1, 1, F), lambda g, be, ba: (be[g], 0, 0)),
            pl.BlockSpec((1, F, D), lambda g, be, ba: (be[g], 0, 0)),
            pl.BlockSpec((1, 1, D), lambda g, be, ba: (be[g], 0, 0)),
        ],
        out_specs=pl.BlockSpec((B, D), lambda g, be, ba: (g, 0)),
    )
    return pl.pallas_call(
        _ffn_body,
        grid_spec=grid_spec,
        out_shape=jax.ShapeDtypeStruct((PTOT, D), jnp.float32),
        compiler_params=pltpu.CompilerParams(
            dimension_semantics=("arbitrary",)),
    )(block_e, block_a, x_sorted, W1, b1.reshape(E, 1, F), W2,
      b2.reshape(E, 1, D))


_BT = 512


def _pair_add_body(g_ref, s_ref, o_ref):
    o_ref[...] = (g_ref[:, :D] * s_ref[:, 0:1] +
                  g_ref[:, D:] * s_ref[:, 1:2])


def _pair_add(g, gate_score):
    return pl.pallas_call(
        _pair_add_body,
        grid=(N // _BT,),
        in_specs=[pl.BlockSpec((_BT, 2 * D), lambda i: (i, 0)),
                  pl.BlockSpec((_BT, K), lambda i: (i, 0))],
        out_specs=pl.BlockSpec((_BT, D), lambda i: (i, 0)),
        out_shape=jax.ShapeDtypeStruct((N, D), jnp.float32),
    )(g, gate_score)


def kernel(inp, w_gate, b_gate, W1, b1, W2, b2):
    gate_score, pos_pair, be, ba = _gate(inp, w_gate, b_gate)
    pe, po = pos_pair[:, 0], pos_pair[:, 1]
    block_e, block_a = be[:, 0], ba[:, 0]
    x_sorted = _make_dispatch()(inp, pe, po)
    y_sorted = _ffn(x_sorted, W1, b1, W2, b2, block_e, block_a)
    g = _make_combine_gather()(y_sorted, pe, po)
    return _pair_add(g, gate_score)


# speedup vs baseline: 1.0055x; 1.0055x over previous
import functools

import jax
import jax.numpy as jnp
from jax import lax
from jax.experimental import pallas as pl
from jax.experimental.pallas import tpu as pltpu
from jax.experimental.pallas import tpu_sc as plsc

E = 8
K = 2
D = 768
F = 3072
N = 2048
S = N * K
B = 256
NB = S // B + E
PTOT = NB * B
FB = 768
NF = F // FB

NC, NS = 2, 16
NW = NC * NS


def _cumsum_rows(x, n):
    c = x
    sh = 1
    while sh < n:
        c = c + jnp.concatenate(
            [jnp.zeros((sh, x.shape[1]), x.dtype), c[:-sh]], axis=0)
        sh *= 2
    return c


def _cumsum_lanes(x, n):
    c = x
    sh = 1
    while sh < n:
        c = c + jnp.concatenate(
            [jnp.zeros((x.shape[0], sh), x.dtype), c[:, :-sh]], axis=1)
        sh *= 2
    return c


def _gate_body(x_ref, wg_ref, bg_ref, gs_ref, pe_ref, po_ref, be_ref, ba_ref):
    logits = jnp.dot(x_ref[...], wg_ref[...],
                     preferred_element_type=jnp.float32) + bg_ref[...]
    col = lax.broadcasted_iota(jnp.int32, (N, E), 1)
    v0 = jnp.max(logits, axis=1, keepdims=True)
    i0 = jnp.min(jnp.where(logits == v0, col, E), axis=1, keepdims=True)
    masked = jnp.where(col == i0, -jnp.inf, logits)
    v1 = jnp.max(masked, axis=1, keepdims=True)
    i1 = jnp.min(jnp.where(masked == v1, col, E), axis=1, keepdims=True)
    e = jnp.exp(v1 - v0)
    s0 = 1.0 / (1.0 + e)
    gs_ref[...] = jnp.concatenate([s0, 1.0 - s0], axis=1)

    oh0 = (col == i0).astype(jnp.int32)
    oh1 = (col == i1).astype(jnp.int32)
    both = oh0 + oh1
    tot = jnp.sum(both, axis=0, keepdims=True)
    cex = _cumsum_rows(both, N) - both
    blocks_per_e = (tot + B - 1) // B
    cumb = _cumsum_lanes(blocks_per_e, E)
    p_off = jnp.concatenate(
        [jnp.zeros((1, 1), jnp.int32), cumb[:, :-1]], axis=1) * B
    rank0 = jnp.sum(jnp.where(col == i0, cex, 0), axis=1, keepdims=True)
    rank1 = jnp.sum(jnp.where(col == i1, cex, 0), axis=1, keepdims=True)
    off0 = jnp.sum(jnp.where(col == i0, p_off, 0), axis=1, keepdims=True)
    off1 = jnp.sum(jnp.where(col == i1, p_off, 0), axis=1, keepdims=True)
    pe_ref[...] = off0 + rank0
    po_ref[...] = off1 + rank1

    num_active = cumb[0, E - 1]
    gcol = lax.broadcasted_iota(jnp.int32, (NB, E), 0)
    be_raw = jnp.sum((gcol >= jnp.broadcast_to(cumb, (NB, E))).astype(
        jnp.int32), axis=1, keepdims=True)
    ecol = lax.broadcasted_iota(jnp.int32, (NB, E), 1)
    last_e = jnp.max(jnp.where(jnp.broadcast_to(tot, (NB, E)) > 0, ecol, 0),
                     axis=1, keepdims=True)
    gid = lax.broadcasted_iota(jnp.int32, (NB, 1), 0)
    be_ref[...] = jnp.where(gid < num_active,
                            jnp.minimum(be_raw, E - 1), last_e)
    ba_ref[...] = (gid < num_active).astype(jnp.int32)


def _gate(inp, w_gate, b_gate):
    return pl.pallas_call(
        _gate_body,
        out_shape=(jax.ShapeDtypeStruct((N, K), jnp.float32),
                   jax.ShapeDtypeStruct((N, 1), jnp.int32),
                   jax.ShapeDtypeStruct((N, 1), jnp.int32),
                   jax.ShapeDtypeStruct((NB, 1), jnp.int32),
                   jax.ShapeDtypeStruct((NB, 1), jnp.int32)),
    )(inp, w_gate, b_gate.reshape(1, E))


_CH = 64


_TPW = N // NW


@functools.cache
def _make_dispatch():
    @functools.partial(
        pl.kernel,
        out_type=jax.ShapeDtypeStruct((PTOT, D), jnp.float32),
        mesh=plsc.VectorSubcoreMesh(core_axis_name="c", subcore_axis_name="s",
                                    num_cores=NC, num_subcores=NS),
        scratch_types=[
            pltpu.VMEM((_TPW,), jnp.int32),
            pltpu.VMEM((_TPW,), jnp.int32),
            pltpu.VMEM((_TPW, D), jnp.float32),
            pltpu.SemaphoreType.DMA,
        ],
    )
    def _dispatch(inp_h, pe_h, po_h, x_h, pe_v, po_v, rows_v, sem):
        wid = lax.axis_index("s") * NC + lax.axis_index("c")
        base = wid * _TPW
        pltpu.sync_copy(inp_h.at[pl.ds(base, _TPW)], rows_v)
        pltpu.sync_copy(pe_h.at[pl.ds(base, _TPW)], pe_v)
        pltpu.sync_copy(po_h.at[pl.ds(base, _TPW)], po_v)
        a = pltpu.async_copy(rows_v, x_h.at[pe_v], sem)
        b = pltpu.async_copy(rows_v, x_h.at[po_v], sem)
        a.wait()
        b.wait()

    return _dispatch


@functools.cache
def _make_combine_gather():
    @functools.partial(
        pl.kernel,
        out_type=jax.ShapeDtypeStruct((N, 2 * D), jnp.float32),
        mesh=plsc.VectorSubcoreMesh(core_axis_name="c", subcore_axis_name="s",
                                    num_cores=NC, num_subcores=NS),
        scratch_types=[
            pltpu.VMEM((_TPW,), jnp.int32),
            pltpu.VMEM((_TPW,), jnp.int32),
            pltpu.VMEM((_TPW, D), jnp.float32),
            pltpu.VMEM((_TPW, D), jnp.float32),
            pltpu.SemaphoreType.DMA,
        ],
    )
    def _cgather(y_h, pe_h, po_h, out_h, pe_v, po_v, a_v, b_v, sem):
        wid = lax.axis_index("s") * NC + lax.axis_index("c")
        tb = wid * _TPW
        pltpu.sync_copy(pe_h.at[pl.ds(tb, _TPW)], pe_v)
        pltpu.sync_copy(po_h.at[pl.ds(tb, _TPW)], po_v)
        a = pltpu.async_copy(y_h.at[pe_v], a_v, sem)
        b = pltpu.async_copy(y_h.at[po_v], b_v, sem)
        a.wait()
        b.wait()
        pltpu.sync_copy(a_v, out_h.at[pl.ds(tb, _TPW), pl.ds(0, D)])
        pltpu.sync_copy(b_v, out_h.at[pl.ds(tb, _TPW), pl.ds(D, D)])

    return _cgather


def _ffn_body(be_ref, ba_ref, x_ref, w1_ref, b1_ref, w2_ref, b2_ref, y_ref):
    g = pl.program_id(0)

    @pl.when(ba_ref[g, 0] == 1)
    def _():
        h = jnp.dot(x_ref[...], w1_ref[0],
                    preferred_element_type=jnp.float32) + b1_ref[0, 0, :]
        y = jnp.dot(jax.nn.gelu(h), w2_ref[0],
                    preferred_element_type=jnp.float32)
        y_ref[...] = y + b2_ref[0, 0, :]


def _ffn(x_sorted, W1, b1, W2, b2, block_e, block_a):
    grid_spec = pltpu.PrefetchScalarGridSpec(
        num_scalar_prefetch=2,
        grid=(NB,),
        in_specs=[
            pl.BlockSpec((B, D), lambda g, be, ba: (g, 0)),
            pl.BlockSpec((1, D, F), lambda g, be, ba: (be[g, 0], 0, 0)),
            pl.BlockSpec((1, 1, F), lambda g, be, ba: (be[g, 0], 0, 0)),
            pl.BlockSpec((1, F, D), lambda g, be, ba: (be[g, 0], 0, 0)),
            pl.BlockSpec((1, 1, D), lambda g, be, ba: (be[g, 0], 0, 0)),
        ],
        out_specs=pl.BlockSpec((B, D), lambda g, be, ba: (g, 0)),
    )
    return pl.pallas_call(
        _ffn_body,
        grid_spec=grid_spec,
        out_shape=jax.ShapeDtypeStruct((PTOT, D), jnp.float32),
        compiler_params=pltpu.CompilerParams(
            dimension_semantics=("arbitrary",),
            vmem_limit_bytes=100 * 1024 * 1024),
    )(block_e, block_a, x_sorted, W1, b1.reshape(E, 1, F), W2,
      b2.reshape(E, 1, D))


_BT = 512


def _pair_add_body(g_ref, s_ref, o_ref):
    o_ref[...] = (g_ref[:, :D] * s_ref[:, 0:1] +
                  g_ref[:, D:] * s_ref[:, 1:2])


def _pair_add(g, gate_score):
    return pl.pallas_call(
        _pair_add_body,
        grid=(N // _BT,),
        in_specs=[pl.BlockSpec((_BT, 2 * D), lambda i: (i, 0)),
                  pl.BlockSpec((_BT, K), lambda i: (i, 0))],
        out_specs=pl.BlockSpec((_BT, D), lambda i: (i, 0)),
        out_shape=jax.ShapeDtypeStruct((N, D), jnp.float32),
    )(g, gate_score)


def kernel(inp, w_gate, b_gate, W1, b1, W2, b2):
    gate_score, pe, po, block_e, block_a = _gate(inp, w_gate, b_gate)
    pe = pe.reshape(N)
    po = po.reshape(N)
    x_sorted = _make_dispatch()(inp, pe, po)
    y_sorted = _ffn(x_sorted, W1, b1, W2, b2, block_e, block_a)
    g = _make_combine_gather()(y_sorted, pe, po)
    return _pair_add(g, gate_score)
